# baseline (speedup 1.0000x reference)
import jax
import jax.numpy as jnp
from jax import lax
from jax.experimental import pallas as pl
from jax.experimental.pallas import tpu as pltpu
from jax.experimental.pallas import tpu_sc as plsc

N = 10000
E = 320000
D = 128
H = 128
GATE = 3 * H

NC = 2
NS = 16
CHUNK = 64
NBUF = 4
IG = 64
NCHUNK = 320
NGROUP = NCHUNK // IG
EPT = NCHUNK * CHUNK
E_PAD = EPT * NS
N_PAD = 10112
ZROWS = N_PAD // NS
RPT = 632
RPT_LAST = N - (NS - 1) * RPT

ROWS_TC = 1000


def _seg_sum_body(x_hbm, h_hbm, srcg_hbm, dstg_hbm, zeros_hbm, aggx_hbm,
                  aggh_hbm, src_v, dst_v, rows, sems, accum_sh):
    cid = lax.axis_index("c")
    sid = lax.axis_index("s")

    pltpu.sync_copy(zeros_hbm.at[pl.ds(sid * ZROWS, ZROWS)],
                    accum_sh.at[pl.ds(sid * ZROWS, ZROWS)])
    plsc.subcore_barrier()

    def gather(m, i):
        @pl.when(cid == 0)
        def _():
            pltpu.async_copy(x_hbm.at[src_v.at[m]], rows[i], sems[i])

        @pl.when(cid != 0)
        def _():
            pltpu.async_copy(h_hbm.at[src_v.at[m]], rows[i], sems[i])

    def group(g, carry):
        pltpu.sync_copy(srcg_hbm.at[sid, pl.ds(g * IG, IG)], src_v)
        pltpu.sync_copy(dstg_hbm.at[sid, pl.ds(g * IG, IG)], dst_v)
        for i in range(NBUF - 1):
            gather(i, i)

        def quad(q, carry2):
            j = NBUF * q
            for i in range(NBUF):
                m = j + i
                nb = (i + NBUF - 1) % NBUF

                @pl.when(m + NBUF - 1 < IG)
                def _():
                    gather(m + NBUF - 1, nb)

                pltpu.make_async_copy(x_hbm.at[src_v.at[m]], rows[i],
                                      sems[i]).wait()
                pltpu.sync_copy(rows[i], accum_sh.at[dst_v.at[m]], add=True)
            return carry2

        lax.fori_loop(0, IG // NBUF, quad, 0)
        return carry

    lax.fori_loop(0, NGROUP, group, 0)
    plsc.subcore_barrier()

    sl = pl.ds(sid * RPT, RPT)
    sl_last = pl.ds((NS - 1) * RPT, RPT_LAST)
    last = sid == NS - 1

    @pl.when(jnp.logical_and(cid == 0, jnp.logical_not(last)))
    def _():
        pltpu.sync_copy(accum_sh.at[sl], aggx_hbm.at[sl])

    @pl.when(jnp.logical_and(cid == 0, last))
    def _():
        pltpu.sync_copy(accum_sh.at[sl_last], aggx_hbm.at[sl_last])

    @pl.when(jnp.logical_and(cid != 0, jnp.logical_not(last)))
    def _():
        pltpu.sync_copy(accum_sh.at[sl], aggh_hbm.at[sl])

    @pl.when(jnp.logical_and(cid != 0, last))
    def _():
        pltpu.sync_copy(accum_sh.at[sl_last], aggh_hbm.at[sl_last])


def _segment_sums(x, h, srcg, dstg, zeros):
    mesh = plsc.VectorSubcoreMesh(core_axis_name="c", subcore_axis_name="s")
    return pl.kernel(
        _seg_sum_body,
        out_type=(jax.ShapeDtypeStruct((N, D), jnp.float32),
                  jax.ShapeDtypeStruct((N, H), jnp.float32)),
        mesh=mesh,
        scratch_types=[
            pltpu.VMEM((IG, CHUNK), jnp.int32),
            pltpu.VMEM((IG, CHUNK), jnp.int32),
            [pltpu.VMEM((CHUNK, D), jnp.float32) for _ in range(NBUF)],
            [pltpu.SemaphoreType.DMA for _ in range(NBUF)],
            pltpu.VMEM_SHARED((N_PAD, D), jnp.float32),
        ],
    )(x, h, srcg, dstg, zeros)


def _root_body(x_ref, h_ref, wx_ref, wh_ref, bx_ref, bh_ref, ox_ref, oh_ref):
    ox_ref[...] = jnp.dot(x_ref[...], wx_ref[...],
                          preferred_element_type=jnp.float32) + bx_ref[...]
    oh_ref[...] = jnp.dot(h_ref[...], wh_ref[...],
                          preferred_element_type=jnp.float32) + bh_ref[...]


def _gru_root(x, h, wx_root, wh_root, bxc, bhc):
    grid = (N // ROWS_TC,)
    row_spec = pl.BlockSpec((ROWS_TC, H), lambda i: (i, 0))
    gate_spec = pl.BlockSpec((ROWS_TC, GATE), lambda i: (i, 0))
    w_spec = pl.BlockSpec((D, GATE), lambda i: (0, 0))
    b_spec = pl.BlockSpec((1, GATE), lambda i: (0, 0))
    return pl.pallas_call(
        _root_body,
        grid=grid,
        in_specs=[row_spec, row_spec, w_spec, w_spec, b_spec, b_spec],
        out_specs=(gate_spec, gate_spec),
        out_shape=(jax.ShapeDtypeStruct((N, GATE), jnp.float32),
                   jax.ShapeDtypeStruct((N, GATE), jnp.float32)),
    )(x, h, wx_root, wh_root, bxc, bhc)


def _gru_body(rx_ref, rh_ref, ax_ref, ah_ref, h_ref, wx_ref, wh_ref,
              out_ref):
    wx = rx_ref[...] + jnp.dot(ax_ref[...], wx_ref[...],
                               preferred_element_type=jnp.float32)
    wh = rh_ref[...] + jnp.dot(ah_ref[...], wh_ref[...],
                               preferred_element_type=jnp.float32)
    r = jax.nn.sigmoid(wx[:, :H] + wh[:, :H])
    z = jax.nn.sigmoid(wx[:, H:2 * H] + wh[:, H:2 * H])
    q = jnp.tanh(wx[:, 2 * H:] + r * wh[:, 2 * H:])
    out_ref[...] = (1.0 - z) * q + z * h_ref[...]


def _gru_dense(rootx, rooth, agg_x, agg_h, h, wx_rel, wh_rel):
    grid = (N // ROWS_TC,)
    row_spec = pl.BlockSpec((ROWS_TC, H), lambda i: (i, 0))
    gate_spec = pl.BlockSpec((ROWS_TC, GATE), lambda i: (i, 0))
    w_spec = pl.BlockSpec((D, GATE), lambda i: (0, 0))
    return pl.pallas_call(
        _gru_body,
        grid=grid,
        in_specs=[gate_spec, gate_spec, row_spec, row_spec, row_spec,
                  w_spec, w_spec],
        out_specs=row_spec,
        out_shape=jax.ShapeDtypeStruct((N, H), jnp.float32),
    )(rootx, rooth, agg_x, agg_h, h, wx_rel, wh_rel)


def kernel(x, edge_index, h, Wx_rel, Wx_root, bx_rel, Wh_rel, Wh_root, bh_rel,
           bias):
    src = edge_index[0].astype(jnp.int32)
    dst = edge_index[1].astype(jnp.int32)
    pad = E_PAD - E
    src_p = jnp.concatenate([src, jnp.zeros((pad,), jnp.int32)])
    dst_p = jnp.concatenate([dst, jnp.full((pad,), N, jnp.int32)])
    srcg = src_p.reshape(NS, NCHUNK, CHUNK)
    dstg = dst_p.reshape(NS, NCHUNK, CHUNK)
    zeros = jnp.zeros((N_PAD, D), jnp.float32)

    bxc = (bx_rel + bias).reshape(1, GATE)
    bhc = bh_rel.reshape(1, GATE)
    agg_x, agg_h = _segment_sums(x, h, srcg, dstg, zeros)
    rootx, rooth = _gru_root(x, h, Wx_root, Wh_root, bxc, bhc)
    return _gru_dense(rootx, rooth, agg_x, agg_h, h, Wx_rel, Wh_rel)


# speedup vs baseline: 1.0014x; 1.0014x over previous
import jax
import jax.numpy as jnp
from jax import lax
from jax.experimental import pallas as pl
from jax.experimental.pallas import tpu as pltpu
from jax.experimental.pallas import tpu_sc as plsc

N = 10000
E = 320000
D = 128
H = 128
GATE = 3 * H

NC = 2
NS = 16
CHUNK = 64
NBUF = 4
IG = 64
NCHUNK = 320
NGROUP = NCHUNK // IG
EPT = NCHUNK * CHUNK
E_PAD = EPT * NS
N_PAD = 10112
ZROWS = N_PAD // NS
RPT = 632
RPT_LAST = N - (NS - 1) * RPT

ROWS_TC = 1000


def _seg_sum_body(x_hbm, h_hbm, srcg_hbm, dstg_hbm, zeros_hbm, aggx_hbm,
                  aggh_hbm, src_v, dst_v, rows, sems, accum_sh):
    cid = lax.axis_index("c")
    sid = lax.axis_index("s")

    pltpu.sync_copy(zeros_hbm.at[pl.ds(sid * ZROWS, ZROWS)],
                    accum_sh.at[pl.ds(sid * ZROWS, ZROWS)])
    plsc.subcore_barrier()

    def gather(m, i):
        @pl.when(cid == 0)
        def _():
            pltpu.async_copy(x_hbm.at[src_v.at[m]], rows[i], sems[i])

        @pl.when(cid != 0)
        def _():
            pltpu.async_copy(h_hbm.at[src_v.at[m]], rows[i], sems[i])

    def group(g, carry):
        pltpu.sync_copy(srcg_hbm.at[sid, pl.ds(g * IG, IG)], src_v)
        pltpu.sync_copy(dstg_hbm.at[sid, pl.ds(g * IG, IG)], dst_v)
        for i in range(NBUF - 1):
            gather(i, i)

        def quad(q, carry2):
            j = NBUF * q
            for i in range(NBUF):
                m = j + i
                nb = (i + NBUF - 1) % NBUF

                @pl.when(m + NBUF - 1 < IG)
                def _():
                    gather(m + NBUF - 1, nb)

                pltpu.make_async_copy(x_hbm.at[src_v.at[m]], rows[i],
                                      sems[i]).wait()
                pltpu.sync_copy(rows[i], accum_sh.at[dst_v.at[m]], add=True)
            return carry2

        lax.fori_loop(0, IG // NBUF, quad, 0)
        return carry

    lax.fori_loop(0, NGROUP, group, 0)
    plsc.subcore_barrier()

    sl = pl.ds(sid * RPT, RPT)
    sl_last = pl.ds((NS - 1) * RPT, RPT_LAST)
    last = sid == NS - 1

    @pl.when(jnp.logical_and(cid == 0, jnp.logical_not(last)))
    def _():
        pltpu.sync_copy(accum_sh.at[sl], aggx_hbm.at[sl])

    @pl.when(jnp.logical_and(cid == 0, last))
    def _():
        pltpu.sync_copy(accum_sh.at[sl_last], aggx_hbm.at[sl_last])

    @pl.when(jnp.logical_and(cid != 0, jnp.logical_not(last)))
    def _():
        pltpu.sync_copy(accum_sh.at[sl], aggh_hbm.at[sl])

    @pl.when(jnp.logical_and(cid != 0, last))
    def _():
        pltpu.sync_copy(accum_sh.at[sl_last], aggh_hbm.at[sl_last])


def _segment_sums(x, h, srcg, dstg, zeros):
    mesh = plsc.VectorSubcoreMesh(core_axis_name="c", subcore_axis_name="s")
    return pl.kernel(
        _seg_sum_body,
        out_type=(jax.ShapeDtypeStruct((N, D), jnp.float32),
                  jax.ShapeDtypeStruct((N, H), jnp.float32)),
        mesh=mesh,
        scratch_types=[
            pltpu.VMEM((IG, CHUNK), jnp.int32),
            pltpu.VMEM((IG, CHUNK), jnp.int32),
            [pltpu.VMEM((CHUNK, D), jnp.float32) for _ in range(NBUF)],
            [pltpu.SemaphoreType.DMA for _ in range(NBUF)],
            pltpu.VMEM_SHARED((N_PAD, D), jnp.float32),
        ],
    )(x, h, srcg, dstg, zeros)


def _root_body(x_ref, h_ref, wx_ref, wh_ref, bx_ref, bh_ref, ox_ref, oh_ref):
    ox_ref[...] = jnp.dot(x_ref[...], wx_ref[...],
                          preferred_element_type=jnp.float32) + bx_ref[...]
    oh_ref[...] = jnp.dot(h_ref[...], wh_ref[...],
                          preferred_element_type=jnp.float32) + bh_ref[...]


def _gru_root(x, h, wx_root, wh_root, bxc, bhc):
    grid = (N // ROWS_TC,)
    row_spec = pl.BlockSpec((ROWS_TC, H), lambda i: (i, 0))
    gate_spec = pl.BlockSpec((ROWS_TC, GATE), lambda i: (i, 0))
    w_spec = pl.BlockSpec((D, GATE), lambda i: (0, 0))
    b_spec = pl.BlockSpec((1, GATE), lambda i: (0, 0))
    return pl.pallas_call(
        _root_body,
        grid=grid,
        in_specs=[row_spec, row_spec, w_spec, w_spec, b_spec, b_spec],
        out_specs=(gate_spec, gate_spec),
        out_shape=(jax.ShapeDtypeStruct((N, GATE), jnp.float32),
                   jax.ShapeDtypeStruct((N, GATE), jnp.float32)),
    )(x, h, wx_root, wh_root, bxc, bhc)


def _gru_body(rx_ref, rh_ref, ax_ref, ah_ref, h_ref, wx_ref, wh_ref,
              out_ref):
    wx = rx_ref[...] + jnp.dot(ax_ref[...], wx_ref[...],
                               preferred_element_type=jnp.float32)
    wh = rh_ref[...] + jnp.dot(ah_ref[...], wh_ref[...],
                               preferred_element_type=jnp.float32)
    r = jax.nn.sigmoid(wx[:, :H] + wh[:, :H])
    z = jax.nn.sigmoid(wx[:, H:2 * H] + wh[:, H:2 * H])
    q = jnp.tanh(wx[:, 2 * H:] + r * wh[:, 2 * H:])
    out_ref[...] = (1.0 - z) * q + z * h_ref[...]


def _gru_dense(rootx, rooth, agg_x, agg_h, h, wx_rel, wh_rel):
    grid = (N // ROWS_TC,)
    row_spec = pl.BlockSpec((ROWS_TC, H), lambda i: (i, 0))
    gate_spec = pl.BlockSpec((ROWS_TC, GATE), lambda i: (i, 0))
    w_spec = pl.BlockSpec((D, GATE), lambda i: (0, 0))
    return pl.pallas_call(
        _gru_body,
        grid=grid,
        in_specs=[gate_spec, gate_spec, row_spec, row_spec, row_spec,
                  w_spec, w_spec],
        out_specs=row_spec,
        out_shape=jax.ShapeDtypeStruct((N, H), jnp.float32),
    )(rootx, rooth, agg_x, agg_h, h, wx_rel, wh_rel)


def kernel(x, edge_index, h, Wx_rel, Wx_root, bx_rel, Wh_rel, Wh_root, bh_rel,
           bias):
    src = edge_index[0].astype(jnp.int32)
    dst = edge_index[1].astype(jnp.int32)
    pad = E_PAD - E
    src_p = jnp.concatenate([src, jnp.zeros((pad,), jnp.int32)])
    dst_p = jnp.concatenate([dst, jnp.full((pad,), N, jnp.int32)])
    srcg = src_p.reshape(NS, NCHUNK, CHUNK)
    dstg = dst_p.reshape(NS, NCHUNK, CHUNK)
    zeros = jnp.zeros((N_PAD, D), jnp.float32)

    bxc = (bx_rel + bias).reshape(1, GATE)
    bhc = bh_rel.reshape(1, GATE)
    rootx, rooth = _gru_root(x, h, Wx_root, Wh_root, bxc, bhc)
    agg_x, agg_h = _segment_sums(x, h, srcg, dstg, zeros)
    return _gru_dense(rootx, rooth, agg_x, agg_h, h, Wx_rel, Wh_rel)


# speedup vs baseline: 1.0220x; 1.0205x over previous
import jax
import jax.numpy as jnp
from jax import lax
from jax.experimental import pallas as pl
from jax.experimental.pallas import tpu as pltpu
from jax.experimental.pallas import tpu_sc as plsc

N = 10000
E = 320000
D = 128
H = 128
GATE = 3 * H

NC = 2
NS = 16
CHUNK = 64
NBUF = 4
IG = 64
NCHUNK = 320
NGROUP = NCHUNK // IG
EPT = NCHUNK * CHUNK
E_PAD = EPT * NS
N_PAD = 10112
ZROWS = N_PAD // NS
RPT = 632
RPT_LAST = N - (NS - 1) * RPT

ROWS_TC = 1000


def _seg_sum_body(x_hbm, h_hbm, srcg_hbm, dstg_hbm, zeros_hbm, aggx_hbm,
                  aggh_hbm, src_v, dst_v, rows, sems, accum_sh):
    cid = lax.axis_index("c")
    sid = lax.axis_index("s")

    pltpu.sync_copy(zeros_hbm.at[pl.ds(sid * ZROWS, ZROWS)],
                    accum_sh.at[pl.ds(sid * ZROWS, ZROWS)])
    plsc.subcore_barrier()

    def gather(m, i):
        @pl.when(cid == 0)
        def _():
            pltpu.async_copy(x_hbm.at[src_v.at[m]], rows[i], sems[i])

        @pl.when(cid != 0)
        def _():
            pltpu.async_copy(h_hbm.at[src_v.at[m]], rows[i], sems[i])

    def group(g, carry):
        pltpu.sync_copy(srcg_hbm.at[sid, pl.ds(g * IG, IG)], src_v)
        pltpu.sync_copy(dstg_hbm.at[sid, pl.ds(g * IG, IG)], dst_v)
        for i in range(NBUF - 1):
            gather(i, i)

        def quad(q, carry2):
            j = NBUF * q
            for i in range(NBUF):
                m = j + i
                nb = (i + NBUF - 1) % NBUF

                @pl.when(m + NBUF - 1 < IG)
                def _():
                    gather(m + NBUF - 1, nb)

                pltpu.make_async_copy(x_hbm.at[src_v.at[m]], rows[i],
                                      sems[i]).wait()
                pltpu.sync_copy(rows[i], accum_sh.at[dst_v.at[m]], add=True)
            return carry2

        lax.fori_loop(0, IG // NBUF, quad, 0)
        return carry

    lax.fori_loop(0, NGROUP, group, 0)
    plsc.subcore_barrier()

    sl = pl.ds(sid * RPT, RPT)
    sl_last = pl.ds((NS - 1) * RPT, RPT_LAST)
    last = sid == NS - 1

    @pl.when(jnp.logical_and(cid == 0, jnp.logical_not(last)))
    def _():
        pltpu.sync_copy(accum_sh.at[sl], aggx_hbm.at[sl])

    @pl.when(jnp.logical_and(cid == 0, last))
    def _():
        pltpu.sync_copy(accum_sh.at[sl_last], aggx_hbm.at[sl_last])

    @pl.when(jnp.logical_and(cid != 0, jnp.logical_not(last)))
    def _():
        pltpu.sync_copy(accum_sh.at[sl], aggh_hbm.at[sl])

    @pl.when(jnp.logical_and(cid != 0, last))
    def _():
        pltpu.sync_copy(accum_sh.at[sl_last], aggh_hbm.at[sl_last])


def _segment_sums(x, h, srcg, dstg, zeros):
    mesh = plsc.VectorSubcoreMesh(core_axis_name="c", subcore_axis_name="s")
    return pl.kernel(
        _seg_sum_body,
        out_type=(jax.ShapeDtypeStruct((N, D), jnp.float32),
                  jax.ShapeDtypeStruct((N, H), jnp.float32)),
        mesh=mesh,
        scratch_types=[
            pltpu.VMEM((IG, CHUNK), jnp.int32),
            pltpu.VMEM((IG, CHUNK), jnp.int32),
            [pltpu.VMEM((CHUNK, D), jnp.float32) for _ in range(NBUF)],
            [pltpu.SemaphoreType.DMA for _ in range(NBUF)],
            pltpu.VMEM_SHARED((N_PAD, D), jnp.float32),
        ],
    )(x, h, srcg, dstg, zeros)


def _gru_body(x_ref, h_ref, ax_ref, ah_ref, wx_ref, wh_ref, bx_ref, bh_ref,
              out_ref):
    xa = jnp.concatenate([x_ref[...], ax_ref[...]], axis=1)
    ha = jnp.concatenate([h_ref[...], ah_ref[...]], axis=1)
    wx = jnp.dot(xa, wx_ref[...], preferred_element_type=jnp.float32)
    wx = wx + bx_ref[...]
    wh = jnp.dot(ha, wh_ref[...], preferred_element_type=jnp.float32)
    wh = wh + bh_ref[...]
    r = jax.nn.sigmoid(wx[:, :H] + wh[:, :H])
    z = jax.nn.sigmoid(wx[:, H:2 * H] + wh[:, H:2 * H])
    q = jnp.tanh(wx[:, 2 * H:] + r * wh[:, 2 * H:])
    out_ref[...] = (1.0 - z) * q + z * h_ref[...]


def _gru_dense(x, h, agg_x, agg_h, wxc, whc, bxc, bhc):
    grid = (N // ROWS_TC,)
    row_spec = pl.BlockSpec((ROWS_TC, H), lambda i: (i, 0))
    w_spec = pl.BlockSpec((D + H, GATE), lambda i: (0, 0))
    b_spec = pl.BlockSpec((1, GATE), lambda i: (0, 0))
    return pl.pallas_call(
        _gru_body,
        grid=grid,
        in_specs=[row_spec, row_spec, row_spec, row_spec,
                  w_spec, w_spec, b_spec, b_spec],
        out_specs=row_spec,
        out_shape=jax.ShapeDtypeStruct((N, H), jnp.float32),
    )(x, h, agg_x, agg_h, wxc, whc, bxc, bhc)


def kernel(x, edge_index, h, Wx_rel, Wx_root, bx_rel, Wh_rel, Wh_root, bh_rel,
           bias):
    src = edge_index[0].astype(jnp.int32)
    dst = edge_index[1].astype(jnp.int32)
    pad = E_PAD - E
    src_p = jnp.concatenate([src, jnp.zeros((pad,), jnp.int32)])
    dst_p = jnp.concatenate([dst, jnp.full((pad,), N, jnp.int32)])
    srcg = src_p.reshape(NS, NCHUNK, CHUNK)
    dstg = dst_p.reshape(NS, NCHUNK, CHUNK)
    zeros = jnp.zeros((N_PAD, D), jnp.float32)

    agg_x, agg_h = _segment_sums(x, h, srcg, dstg, zeros)

    wxc = jnp.concatenate([Wx_root, Wx_rel], axis=0)
    whc = jnp.concatenate([Wh_root, Wh_rel], axis=0)
    bxc = (bx_rel + bias).reshape(1, GATE)
    bhc = bh_rel.reshape(1, GATE)
    return _gru_dense(x, h, agg_x, agg_h, wxc, whc, bxc, bhc)


# speedup vs baseline: 1.0238x; 1.0018x over previous
import jax
import jax.numpy as jnp
from jax import lax
from jax.experimental import pallas as pl
from jax.experimental.pallas import tpu as pltpu
from jax.experimental.pallas import tpu_sc as plsc

N = 10000
E = 320000
D = 128
H = 128
GATE = 3 * H

NC = 2
NS = 16
CHUNK = 64
NBUF = 4
IG = 64
NCHUNK = 320
NGROUP = NCHUNK // IG
EPT = NCHUNK * CHUNK
E_PAD = EPT * NS
N_PAD = 10112
ZROWS = N_PAD // NS
RPT = 632
RPT_LAST = N - (NS - 1) * RPT

ROWS_TC = 1000


def _seg_sum_body(x_hbm, h_hbm, srcg_hbm, dstg_hbm, zeros_hbm, aggx_hbm,
                  aggh_hbm, src_v, dst_v, rows, sems, accum_sh):
    cid = lax.axis_index("c")
    sid = lax.axis_index("s")

    def gather(m, i):
        @pl.when(cid == 0)
        def _():
            pltpu.async_copy(x_hbm.at[src_v.at[m]], rows[i], sems[i])

        @pl.when(cid != 0)
        def _():
            pltpu.async_copy(h_hbm.at[src_v.at[m]], rows[i], sems[i])

    def stage_and_prime(g):
        pltpu.sync_copy(srcg_hbm.at[sid, pl.ds(g * IG, IG)], src_v)
        pltpu.sync_copy(dstg_hbm.at[sid, pl.ds(g * IG, IG)], dst_v)
        for i in range(NBUF - 1):
            gather(i, i)

    stage_and_prime(0)
    pltpu.sync_copy(zeros_hbm.at[pl.ds(sid * ZROWS, ZROWS)],
                    accum_sh.at[pl.ds(sid * ZROWS, ZROWS)])
    plsc.subcore_barrier()

    def group(g, carry):
        @pl.when(g > 0)
        def _():
            stage_and_prime(g)

        def quad(q, carry2):
            j = NBUF * q
            for i in range(NBUF):
                m = j + i
                nb = (i + NBUF - 1) % NBUF

                @pl.when(m + NBUF - 1 < IG)
                def _():
                    gather(m + NBUF - 1, nb)

                pltpu.make_async_copy(x_hbm.at[src_v.at[m]], rows[i],
                                      sems[i]).wait()
                pltpu.sync_copy(rows[i], accum_sh.at[dst_v.at[m]], add=True)
            return carry2

        lax.fori_loop(0, IG // NBUF, quad, 0)
        return carry

    lax.fori_loop(0, NGROUP, group, 0)
    plsc.subcore_barrier()

    sl = pl.ds(sid * RPT, RPT)
    sl_last = pl.ds((NS - 1) * RPT, RPT_LAST)
    last = sid == NS - 1

    @pl.when(jnp.logical_and(cid == 0, jnp.logical_not(last)))
    def _():
        pltpu.sync_copy(accum_sh.at[sl], aggx_hbm.at[sl])

    @pl.when(jnp.logical_and(cid == 0, last))
    def _():
        pltpu.sync_copy(accum_sh.at[sl_last], aggx_hbm.at[sl_last])

    @pl.when(jnp.logical_and(cid != 0, jnp.logical_not(last)))
    def _():
        pltpu.sync_copy(accum_sh.at[sl], aggh_hbm.at[sl])

    @pl.when(jnp.logical_and(cid != 0, last))
    def _():
        pltpu.sync_copy(accum_sh.at[sl_last], aggh_hbm.at[sl_last])


def _segment_sums(x, h, srcg, dstg, zeros):
    mesh = plsc.VectorSubcoreMesh(core_axis_name="c", subcore_axis_name="s")
    return pl.kernel(
        _seg_sum_body,
        out_type=(jax.ShapeDtypeStruct((N, D), jnp.float32),
                  jax.ShapeDtypeStruct((N, H), jnp.float32)),
        mesh=mesh,
        scratch_types=[
            pltpu.VMEM((IG, CHUNK), jnp.int32),
            pltpu.VMEM((IG, CHUNK), jnp.int32),
            [pltpu.VMEM((CHUNK, D), jnp.float32) for _ in range(NBUF)],
            [pltpu.SemaphoreType.DMA for _ in range(NBUF)],
            pltpu.VMEM_SHARED((N_PAD, D), jnp.float32),
        ],
    )(x, h, srcg, dstg, zeros)


def _gru_body(x_ref, h_ref, ax_ref, ah_ref, wx_ref, wh_ref, bx_ref, bh_ref,
              out_ref):
    xa = jnp.concatenate([x_ref[...], ax_ref[...]], axis=1)
    ha = jnp.concatenate([h_ref[...], ah_ref[...]], axis=1)
    wx = jnp.dot(xa, wx_ref[...], preferred_element_type=jnp.float32)
    wx = wx + bx_ref[...]
    wh = jnp.dot(ha, wh_ref[...], preferred_element_type=jnp.float32)
    wh = wh + bh_ref[...]
    r = jax.nn.sigmoid(wx[:, :H] + wh[:, :H])
    z = jax.nn.sigmoid(wx[:, H:2 * H] + wh[:, H:2 * H])
    q = jnp.tanh(wx[:, 2 * H:] + r * wh[:, 2 * H:])
    out_ref[...] = (1.0 - z) * q + z * h_ref[...]


def _gru_dense(x, h, agg_x, agg_h, wxc, whc, bxc, bhc):
    grid = (N // ROWS_TC,)
    row_spec = pl.BlockSpec((ROWS_TC, H), lambda i: (i, 0))
    w_spec = pl.BlockSpec((D + H, GATE), lambda i: (0, 0))
    b_spec = pl.BlockSpec((1, GATE), lambda i: (0, 0))
    return pl.pallas_call(
        _gru_body,
        grid=grid,
        in_specs=[row_spec, row_spec, row_spec, row_spec,
                  w_spec, w_spec, b_spec, b_spec],
        out_specs=row_spec,
        out_shape=jax.ShapeDtypeStruct((N, H), jnp.float32),
    )(x, h, agg_x, agg_h, wxc, whc, bxc, bhc)


def kernel(x, edge_index, h, Wx_rel, Wx_root, bx_rel, Wh_rel, Wh_root, bh_rel,
           bias):
    src = edge_index[0].astype(jnp.int32)
    dst = edge_index[1].astype(jnp.int32)
    pad = E_PAD - E
    src_p = jnp.concatenate([src, jnp.zeros((pad,), jnp.int32)])
    dst_p = jnp.concatenate([dst, jnp.full((pad,), N, jnp.int32)])
    srcg = src_p.reshape(NS, NCHUNK, CHUNK)
    dstg = dst_p.reshape(NS, NCHUNK, CHUNK)
    zeros = jnp.zeros((N_PAD, D), jnp.float32)

    agg_x, agg_h = _segment_sums(x, h, srcg, dstg, zeros)

    wxc = jnp.concatenate([Wx_root, Wx_rel], axis=0)
    whc = jnp.concatenate([Wh_root, Wh_rel], axis=0)
    bxc = (bx_rel + bias).reshape(1, GATE)
    bhc = bh_rel.reshape(1, GATE)
    return _gru_dense(x, h, agg_x, agg_h, wxc, whc, bxc, bhc)
